# baseline (speedup 1.0000x reference)
import functools

import jax
import jax.numpy as jnp
from jax import lax
from jax.experimental import pallas as pl
from jax.experimental.pallas import tpu as pltpu
from jax.experimental.pallas import tpu_sc as plsc

B, N, D = 4, 50000, 128
K = 25000


_SBLK = 2000


def _score_body(x_ref, w_ref, b_ref, o_ref):
    xb = x_ref[0].astype(jnp.bfloat16)
    wb = w_ref[...].astype(jnp.bfloat16)
    logits = jnp.dot(xb, wb, preferred_element_type=jnp.float32)
    o_ref[0, 0, :] = jax.nn.sigmoid(logits[:, 0] + b_ref[0, 0])


def _scores(x, W, b):
    nblk = N // _SBLK
    grid = (B, nblk)
    out = pl.pallas_call(
        _score_body,
        grid=grid,
        in_specs=[
            pl.BlockSpec((1, _SBLK, D), lambda i, j: (i, j, 0)),
            pl.BlockSpec((D, D), lambda i, j: (0, 0)),
            pl.BlockSpec(memory_space=pltpu.SMEM),
        ],
        out_specs=pl.BlockSpec((1, 1, _SBLK), lambda i, j: (i * nblk + j, 0, 0)),
        out_shape=jax.ShapeDtypeStruct((B * nblk, 1, _SBLK), jnp.float32),
    )(x, jnp.zeros((D, D), jnp.float32).at[:, 0].set(W[0]), b.reshape(1, 1))
    return out.reshape(B, N)


_NW = 32
_KPAD = 25088
_RPW = _KPAD // _NW
_CH = 98
_NCH = _RPW // _CH


def _gather_body(x_hbm, idx_hbm, out_hbm, idx_v, rows_v, sem):
    wid = lax.axis_index("s") * 2 + lax.axis_index("c")
    for b in range(B):
        pltpu.sync_copy(idx_hbm.at[b, pl.ds(wid * _NCH, _NCH), :], idx_v)
        for c in range(_NCH):
            pltpu.async_copy(
                x_hbm.at[idx_v.at[c]],
                rows_v.at[pl.ds(c * _CH, _CH)],
                sem,
            )
        for c in range(_NCH):
            pltpu.make_async_copy(
                x_hbm.at[idx_v.at[c]],
                rows_v.at[pl.ds(c * _CH, _CH)],
                sem,
            ).wait()
        pltpu.sync_copy(rows_v, out_hbm.at[b, pl.ds(wid * _RPW, _RPW), :])


def _gather_rows(x, idx_pad):
    x2d = x.reshape(B * N, D)
    offs = (jnp.arange(B, dtype=jnp.int32) * N)[:, None]
    idx3 = (idx_pad + offs).reshape(B, _NW * _NCH, _CH)
    mesh = plsc.VectorSubcoreMesh(core_axis_name="c", subcore_axis_name="s")
    return pl.kernel(
        _gather_body,
        out_type=jax.ShapeDtypeStruct((B, _KPAD, D), jnp.float32),
        mesh=mesh,
        scratch_types=[
            pltpu.VMEM((_NCH, _CH), jnp.int32),
            pltpu.VMEM((_RPW, D), jnp.float32),
            pltpu.SemaphoreType.DMA,
        ],
    )(x2d, idx3)


_CBLK = 1000


def _scale_body(r_ref, v_ref, o_ref):
    o_ref[0] = r_ref[0] * v_ref[0, 0][:, None]


def _scale(raw_pad, vals):
    nblk = K // _CBLK
    grid = (B, nblk)
    vals3 = vals[:, :K].reshape(B * nblk, 1, _CBLK)
    return pl.pallas_call(
        _scale_body,
        grid=grid,
        in_specs=[
            pl.BlockSpec((1, _CBLK, D), lambda i, j: (i, j, 0)),
            pl.BlockSpec((1, 1, _CBLK), lambda i, j: (i * nblk + j, 0, 0)),
        ],
        out_specs=pl.BlockSpec((1, _CBLK, D), lambda i, j: (i, j, 0)),
        out_shape=jax.ShapeDtypeStruct((B, K, D), jnp.float32),
    )(raw_pad, vals3)


def kernel(x, W, b):
    scores = _scores(x, W, b)
    top_vals, top_idx = lax.top_k(scores, K)
    idx_pad = jnp.pad(top_idx, ((0, 0), (0, _KPAD - K)))
    raw = _gather_rows(x, idx_pad.astype(jnp.int32))
    return _scale(raw, top_vals)
